# Initial kernel scaffold; baseline (speedup 1.0000x reference)
import jax
import jax.numpy as jnp
from jax.experimental import pallas as pl


def kernel(indices, table):
    raise NotImplementedError("write your pallas kernel here")


# speedup vs baseline: 1.5649x; 1.5649x over previous
import functools

import jax
import jax.numpy as jnp
from jax import lax
from jax.experimental import pallas as pl
from jax.experimental.pallas import tpu as pltpu
from jax.experimental.pallas import tpu_sc as plsc

_BATCH = 16384
_HIST = 50
_DIM = 64
_NB = _BATCH * _HIST

_INFO = plsc.get_sparse_core_info()
_NC = _INFO.num_cores
_NS = _INFO.num_subcores
_NW = _NC * _NS
_ROWS_PER_W = _NB // _NW
_CHUNK = 128
_NCHUNK = _ROWS_PER_W // _CHUNK


def _embed_kernel(idx_hbm, table_hbm, out_hbm, idx_v, rows_v, sem):
    wid = lax.axis_index("s") * _NC + lax.axis_index("c")
    wbase = wid * _ROWS_PER_W

    def body(i, carry):
        base = wbase + i * _CHUNK
        pltpu.sync_copy(idx_hbm.at[pl.ds(base, _CHUNK)], idx_v)
        pltpu.async_copy(table_hbm.at[idx_v], rows_v, sem).wait()
        pltpu.sync_copy(rows_v, out_hbm.at[pl.ds(base, _CHUNK)])
        return carry

    lax.fori_loop(0, _NCHUNK, body, 0)


@jax.jit
def _embed(indices_flat, table):
    mesh = plsc.VectorSubcoreMesh(core_axis_name="c", subcore_axis_name="s")
    run = functools.partial(
        pl.kernel,
        mesh=mesh,
        out_type=jax.ShapeDtypeStruct((_NB, _DIM), jnp.float32),
        scratch_types=[
            pltpu.VMEM((_CHUNK,), jnp.int32),
            pltpu.VMEM((_CHUNK, _DIM), jnp.float32),
            pltpu.SemaphoreType.DMA,
        ],
        compiler_params=pltpu.CompilerParams(use_tc_tiling_on_sc=False),
    )(_embed_kernel)
    return run(indices_flat, table)


def kernel(indices, table):
    out = _embed(indices.reshape(-1), table)
    return out.reshape(*indices.shape, table.shape[1])


# speedup vs baseline: 1.8741x; 1.1976x over previous
import functools

import jax
import jax.numpy as jnp
from jax import lax
from jax.experimental import pallas as pl
from jax.experimental.pallas import tpu as pltpu
from jax.experimental.pallas import tpu_sc as plsc

_BATCH = 16384
_HIST = 50
_DIM = 64
_NB = _BATCH * _HIST

_INFO = plsc.get_sparse_core_info()
_NC = _INFO.num_cores
_NS = _INFO.num_subcores
_NW = _NC * _NS
_ROWS_PER_W = _NB // _NW
_CHUNK = 128
_K = 5
_GROUP = _CHUNK * _K
_NGRP = _ROWS_PER_W // _GROUP
_NBUF = 2


def _embed_kernel(idx_hbm, table_hbm, out_hbm, idx_v, rows0, rows1, gsem,
                  wsem0, wsem1):
    wid = lax.axis_index("s") * _NC + lax.axis_index("c")
    wbase = wid * _ROWS_PER_W
    rows = (rows0, rows1)
    wsems = (wsem0, wsem1)

    pltpu.sync_copy(idx_hbm.at[pl.ds(wbase, _ROWS_PER_W)], idx_v)

    def fire_and_drain(g, buf):
        copies = [
            pltpu.async_copy(
                table_hbm.at[idx_v.at[pl.ds(g * _GROUP + j * _CHUNK, _CHUNK)]],
                buf.at[pl.ds(j * _CHUNK, _CHUNK)],
                gsem,
            )
            for j in range(_K)
        ]
        for c in copies:
            c.wait()

    def start_writeback(g, buf, sem):
        pltpu.async_copy(buf, out_hbm.at[pl.ds(wbase + g * _GROUP, _GROUP)], sem)

    for b in range(_NBUF):
        fire_and_drain(b, rows[b])
        start_writeback(b, rows[b], wsems[b])

    def body(gg):
        for b in range(_NBUF):
            g = gg + b
            pltpu.make_async_copy(
                rows[b], out_hbm.at[pl.ds(wbase, _GROUP)], wsems[b]).wait()
            fire_and_drain(g, rows[b])
            start_writeback(g, rows[b], wsems[b])

    pl.loop(_NBUF, _NGRP, step=_NBUF)(body)

    for b in range(_NBUF):
        pltpu.make_async_copy(
            rows[b], out_hbm.at[pl.ds(wbase, _GROUP)], wsems[b]).wait()


@jax.jit
def _embed(indices_flat, table):
    mesh = plsc.VectorSubcoreMesh(core_axis_name="c", subcore_axis_name="s")
    run = functools.partial(
        pl.kernel,
        mesh=mesh,
        out_type=jax.ShapeDtypeStruct((_NB, _DIM), jnp.float32),
        scratch_types=[
            pltpu.VMEM((_ROWS_PER_W,), jnp.int32),
            pltpu.VMEM((_GROUP, _DIM), jnp.float32),
            pltpu.VMEM((_GROUP, _DIM), jnp.float32),
            pltpu.SemaphoreType.DMA,
            pltpu.SemaphoreType.DMA,
            pltpu.SemaphoreType.DMA,
        ],
        compiler_params=pltpu.CompilerParams(use_tc_tiling_on_sc=False),
    )(_embed_kernel)
    return run(indices_flat, table)


def kernel(indices, table):
    out = _embed(indices.reshape(-1), table)
    return out.reshape(*indices.shape, table.shape[1])
